# Initial kernel scaffold; baseline (speedup 1.0000x reference)
import functools

import jax
import jax.numpy as jnp
from jax import lax
from jax.experimental import pallas as pl
from jax.experimental.pallas import tpu as pltpu
from jax.experimental.pallas import tpu_sc as plsc

B = 16384
D = 64
C = 20
N = 100
K = C + N
RBLK = 128
NVREG = D // 16


@functools.lru_cache(maxsize=None)
def _build_sc_kernel():
  info = plsc.get_sparse_core_info()
  nc, ns = info.num_cores, info.num_subcores
  nw = nc * ns
  rpw = B // nw
  nblk = rpw // RBLK
  mesh = plsc.VectorSubcoreMesh(core_axis_name="c", subcore_axis_name="s")

  @functools.partial(
      pl.kernel,
      out_type=(
          jax.ShapeDtypeStruct((B, D), jnp.float32),
          jax.ShapeDtypeStruct((B, D), jnp.float32),
          jax.ShapeDtypeStruct((B, D), jnp.float32),
      ),
      mesh=mesh,
      scratch_types=(
          pltpu.VMEM((RBLK,), jnp.int32),
          pltpu.VMEM((RBLK, K), jnp.int32),
          pltpu.VMEM((RBLK, D), jnp.float32),
          pltpu.VMEM((K, D), jnp.float32),
          pltpu.VMEM((K, D), jnp.float32),
          pltpu.VMEM((RBLK, D), jnp.float32),
          pltpu.VMEM((RBLK, D), jnp.float32),
          pltpu.SemaphoreType.DMA,
          pltpu.SemaphoreType.DMA,
      ),
  )
  def sc_kernel(center_hbm, idx_hbm, in_emb_hbm, out_emb_hbm,
                crows_o, ctxsum_o, negsum_o,
                cidx_v, idx_v, crows_v, buf_a, buf_b,
                ctxsum_v, negsum_v, sem_a, sem_b):
    wid = lax.axis_index("s") * nc + lax.axis_index("c")

    def issue(b, buf, sem):
      pltpu.make_async_copy(out_emb_hbm.at[idx_v.at[b]], buf, sem).start()

    def drain(buf, sem):
      pltpu.make_async_copy(out_emb_hbm.at[idx_v.at[0]], buf, sem).wait()

    def reduce_store(buf, b):
      for k in range(NVREG):
        sl = pl.ds(16 * k, 16)
        acc_c = buf[0, sl]
        for j in range(1, C):
          acc_c = acc_c + buf[j, sl]
        acc_n = buf[C, sl]
        for j in range(C + 1, K):
          acc_n = acc_n + buf[j, sl]
        ctxsum_v[b, sl] = acc_c
        negsum_v[b, sl] = acc_n

    def block(blk, carry):
      base = pl.multiple_of(wid * rpw + blk * RBLK, RBLK)
      pltpu.sync_copy(center_hbm.at[pl.ds(base, RBLK)], cidx_v)
      pltpu.sync_copy(idx_hbm.at[pl.ds(base, RBLK), :], idx_v)
      pltpu.async_copy(in_emb_hbm.at[cidx_v], crows_v, sem_a).wait()
      pltpu.sync_copy(crows_v, crows_o.at[pl.ds(base, RBLK), :])

      issue(0, buf_a, sem_a)

      def pair(g, c2):
        b0 = g * 2
        issue(b0 + 1, buf_b, sem_b)
        drain(buf_a, sem_a)
        reduce_store(buf_a, b0)

        @pl.when(b0 + 2 < RBLK)
        def _():
          issue(b0 + 2, buf_a, sem_a)

        drain(buf_b, sem_b)
        reduce_store(buf_b, b0 + 1)
        return c2

      lax.fori_loop(0, RBLK // 2, pair, 0)

      pltpu.sync_copy(ctxsum_v, ctxsum_o.at[pl.ds(base, RBLK), :])
      pltpu.sync_copy(negsum_v, negsum_o.at[pl.ds(base, RBLK), :])
      return carry

    lax.fori_loop(0, nblk, block, 0)

  return sc_kernel


def _tc_score(crows, ctxsum, negsum):
  bt = 2048

  def body(c_ref, cs_ref, ns_ref, o_ref):
    c = c_ref[...]
    s_ctx = jnp.sum(cs_ref[...] * c, axis=1)
    s_neg = jnp.sum(ns_ref[...] * c, axis=1)
    o_ref[...] = -(jax.nn.log_sigmoid(s_ctx) + jax.nn.log_sigmoid(-s_neg))

  return pl.pallas_call(
      body,
      grid=(B // bt,),
      in_specs=[pl.BlockSpec((bt, D), lambda i: (i, 0))] * 3,
      out_specs=pl.BlockSpec((bt,), lambda i: (i,)),
      out_shape=jax.ShapeDtypeStruct((B,), jnp.float32),
  )(crows, ctxsum, negsum)


def kernel(center_word_label, context_words_labels, neg_words_labels,
           input_embedding, output_embedding):
  idx_all = jnp.concatenate(
      [context_words_labels.astype(jnp.int32),
       neg_words_labels.astype(jnp.int32)], axis=1)
  center = center_word_label.astype(jnp.int32)
  crows, ctxsum, negsum = _build_sc_kernel()(
      center, idx_all, input_embedding, output_embedding)
  return _tc_score(crows, ctxsum, negsum)


# speedup vs baseline: 9.0141x; 9.0141x over previous
import functools

import jax
import jax.numpy as jnp
from jax import lax
from jax.experimental import pallas as pl
from jax.experimental.pallas import tpu as pltpu
from jax.experimental.pallas import tpu_sc as plsc

B = 16384
D = 64
C = 20
N = 100
K = C + N
RBLK = 128
NVREG = D // 16


@functools.lru_cache(maxsize=None)
def _build_sc_kernel():
  info = plsc.get_sparse_core_info()
  nc, ns = info.num_cores, info.num_subcores
  nw = nc * ns
  rpw = B // nw
  nblk = rpw // RBLK
  mesh = plsc.VectorSubcoreMesh(core_axis_name="c", subcore_axis_name="s")

  @functools.partial(
      pl.kernel,
      out_type=(
          jax.ShapeDtypeStruct((B, D), jnp.float32),
          jax.ShapeDtypeStruct((B, D), jnp.float32),
          jax.ShapeDtypeStruct((B, D), jnp.float32),
      ),
      mesh=mesh,
      compiler_params=pltpu.CompilerParams(use_tc_tiling_on_sc=False),
      scratch_types=(
          pltpu.VMEM((RBLK,), jnp.int32),
          pltpu.VMEM((RBLK, K), jnp.int32),
          pltpu.VMEM((RBLK, D), jnp.float32),
          pltpu.VMEM((K, D), jnp.float32),
          pltpu.VMEM((K, D), jnp.float32),
          pltpu.VMEM((RBLK, D), jnp.float32),
          pltpu.VMEM((RBLK, D), jnp.float32),
          pltpu.SemaphoreType.DMA,
          pltpu.SemaphoreType.DMA,
      ),
  )
  def sc_kernel(center_hbm, idx_hbm, in_emb_hbm, out_emb_hbm,
                crows_o, ctxsum_o, negsum_o,
                cidx_v, idx_v, crows_v, buf_a, buf_b,
                ctxsum_v, negsum_v, sem_a, sem_b):
    wid = lax.axis_index("s") * nc + lax.axis_index("c")

    def issue(b, buf, sem):
      pltpu.make_async_copy(out_emb_hbm.at[idx_v.at[b]], buf, sem).start()

    def drain(buf, sem):
      pltpu.make_async_copy(out_emb_hbm.at[idx_v.at[0]], buf, sem).wait()

    def reduce_store(buf, b):
      for k in range(NVREG):
        sl = pl.ds(16 * k, 16)
        acc_c = buf[0, sl]
        for j in range(1, C):
          acc_c = acc_c + buf[j, sl]
        acc_n = buf[C, sl]
        for j in range(C + 1, K):
          acc_n = acc_n + buf[j, sl]
        ctxsum_v[b, sl] = acc_c
        negsum_v[b, sl] = acc_n

    def block(blk, carry):
      base = pl.multiple_of(wid * rpw + blk * RBLK, RBLK)
      pltpu.sync_copy(center_hbm.at[pl.ds(base, RBLK)], cidx_v)
      pltpu.sync_copy(idx_hbm.at[pl.ds(base, RBLK), :], idx_v)
      pltpu.async_copy(in_emb_hbm.at[cidx_v], crows_v, sem_a).wait()
      pltpu.sync_copy(crows_v, crows_o.at[pl.ds(base, RBLK), :])

      issue(0, buf_a, sem_a)

      def pair(g, c2):
        b0 = g * 2
        issue(b0 + 1, buf_b, sem_b)
        drain(buf_a, sem_a)
        reduce_store(buf_a, b0)

        @pl.when(b0 + 2 < RBLK)
        def _():
          issue(b0 + 2, buf_a, sem_a)

        drain(buf_b, sem_b)
        reduce_store(buf_b, b0 + 1)
        return c2

      lax.fori_loop(0, RBLK // 2, pair, 0)

      pltpu.sync_copy(ctxsum_v, ctxsum_o.at[pl.ds(base, RBLK), :])
      pltpu.sync_copy(negsum_v, negsum_o.at[pl.ds(base, RBLK), :])
      return carry

    lax.fori_loop(0, nblk, block, 0)

  return sc_kernel


def _tc_score(crows, ctxsum, negsum):
  bt = 2048

  def body(c_ref, cs_ref, ns_ref, o_ref):
    c = c_ref[...]
    s_ctx = jnp.sum(cs_ref[...] * c, axis=1)
    s_neg = jnp.sum(ns_ref[...] * c, axis=1)
    o_ref[...] = -(jax.nn.log_sigmoid(s_ctx) + jax.nn.log_sigmoid(-s_neg))

  return pl.pallas_call(
      body,
      grid=(B // bt,),
      in_specs=[pl.BlockSpec((bt, D), lambda i: (i, 0))] * 3,
      out_specs=pl.BlockSpec((bt,), lambda i: (i,)),
      out_shape=jax.ShapeDtypeStruct((B,), jnp.float32),
  )(crows, ctxsum, negsum)


def kernel(center_word_label, context_words_labels, neg_words_labels,
           input_embedding, output_embedding):
  idx_all = jnp.concatenate(
      [context_words_labels.astype(jnp.int32),
       neg_words_labels.astype(jnp.int32)], axis=1)
  center = center_word_label.astype(jnp.int32)
  crows, ctxsum, negsum = _build_sc_kernel()(
      center, idx_all, input_embedding, output_embedding)
  return _tc_score(crows, ctxsum, negsum)
